# Initial kernel scaffold; baseline (speedup 1.0000x reference)
import jax
import jax.numpy as jnp
from jax.experimental import pallas as pl


def kernel(x, embed):
    raise NotImplementedError("write your pallas kernel here")


# speedup vs baseline: 1.7282x; 1.7282x over previous
import functools

import jax
import jax.numpy as jnp
from jax.experimental import pallas as pl
from jax.experimental.pallas import tpu as pltpu

B, N, DIM = 16, 1024, 256
BN = B * N
K = 1024
M = 512
NB = BN // M


def _tc_body(x_ref, e_ref, dist_ref, idx_ref, q_ref):
    x = x_ref[...]
    e = e_ref[...]
    cross = jax.lax.dot_general(
        x, e, (((1,), (1,)), ((), ())), preferred_element_type=jnp.float32
    )
    x_sq = jnp.sum(x * x, axis=1, keepdims=True)
    e_sq = jnp.sum(e * e, axis=1)[None, :]
    dist = x_sq + e_sq - 2.0 * cross
    dist_ref[...] = dist
    idx = jnp.argmin(dist, axis=1).astype(jnp.int32)
    idx_ref[...] = idx.reshape(1, 1, M)
    onehot = (
        jax.lax.broadcasted_iota(jnp.int32, (M, K), 1) == idx[:, None]
    ).astype(jnp.float32)
    q_ref[...] = jax.lax.dot_general(
        onehot, e, (((1,), (0,)), ((), ())), preferred_element_type=jnp.float32
    )


def kernel(x, embed):
    xf = x.reshape(BN, DIM)
    e = embed.reshape(K, DIM)
    dist, idx3, q = pl.pallas_call(
        _tc_body,
        grid=(NB,),
        in_specs=[
            pl.BlockSpec((M, DIM), lambda i: (i, 0)),
            pl.BlockSpec((K, DIM), lambda i: (0, 0)),
        ],
        out_specs=[
            pl.BlockSpec((M, K), lambda i: (i, 0)),
            pl.BlockSpec((1, 1, M), lambda i: (i, 0, 0)),
            pl.BlockSpec((M, DIM), lambda i: (i, 0)),
        ],
        out_shape=[
            jax.ShapeDtypeStruct((BN, K), jnp.float32),
            jax.ShapeDtypeStruct((NB, 1, M), jnp.int32),
            jax.ShapeDtypeStruct((BN, DIM), jnp.float32),
        ],
    )(xf, e)
    return q.reshape(BN, 1, DIM), idx3.reshape(BN), dist
